# Initial kernel scaffold; baseline (speedup 1.0000x reference)
import jax
import jax.numpy as jnp
from jax.experimental import pallas as pl


def kernel(ego_embed, y_hat, lane_vectors, edge_attr, edge_index_lane, edge_index_agent, W_fc, b_fc, W_msg, b_msg, W_upd, W_dec, b_dec, W_pi, b_pi):
    raise NotImplementedError("write your pallas kernel here")


# speedup vs baseline: 2.9669x; 2.9669x over previous
import functools

import jax
import jax.numpy as jnp
from jax import lax
from jax.experimental import pallas as pl
from jax.experimental.pallas import tpu as pltpu
from jax.experimental.pallas import tpu_sc as plsc

EMBED = 64
RADIUS2 = 2500.0
NC, NS = 2, 16
NW = NC * NS
CHUNK = 128
GROUPS = CHUNK // 16


def _round_up(x, m):
    return (x + m - 1) // m * m


def _tc_pre(lanes8, Wm8, b_msg, ego, W_fc, b_fc):
    Lp = lanes8.shape[0]
    N = ego.shape[0]

    def body(lanes_ref, wm_ref, bm_ref, ego_ref, wfc_ref, bfc_ref,
             lane2_ref, h0_ref):
        lane2_ref[...] = (
            jnp.dot(lanes_ref[...], wm_ref[...],
                    preferred_element_type=jnp.float32) + bm_ref[...][None, :])
        h0_ref[...] = (
            jnp.dot(ego_ref[...], wfc_ref[...],
                    preferred_element_type=jnp.float32) + bfc_ref[...][None, :])

    return pl.pallas_call(
        body,
        out_shape=(jax.ShapeDtypeStruct((Lp, EMBED), jnp.float32),
                   jax.ShapeDtypeStruct((N, EMBED), jnp.float32)),
    )(lanes8, Wm8, b_msg, ego, W_fc, b_fc)


def _sc_agg(lane2, eil, eia, eax, eay, w23, nseg):
    acc_len = nseg * EMBED
    e_pad = eil.shape[0]
    epw = e_pad // NW
    nchunk = epw // CHUNK
    mesh = plsc.VectorSubcoreMesh(core_axis_name="c", subcore_axis_name="s",
                                  num_cores=NC, num_subcores=NS)

    @functools.partial(
        pl.kernel,
        out_type=jax.ShapeDtypeStruct((NW, acc_len), jnp.float32),
        mesh=mesh,
        compiler_params=pltpu.CompilerParams(needs_layout_passes=False,
                                             use_tc_tiling_on_sc=False),
        scratch_types=[
            pltpu.VMEM((acc_len,), jnp.float32),
            pltpu.VMEM((CHUNK,), jnp.int32),
            pltpu.VMEM((CHUNK,), jnp.int32),
            pltpu.VMEM((CHUNK,), jnp.float32),
            pltpu.VMEM((CHUNK,), jnp.float32),
            pltpu.VMEM((CHUNK, EMBED), jnp.float32),
            pltpu.VMEM((128,), jnp.float32),
            pltpu.SemaphoreType.DMA,
        ],
    )
    def k(lane2_h, eil_h, eia_h, eax_h, eay_h, w23_h, out_h,
          acc, il, ia, ax, ay, g, wv, sem):
        wid = lax.axis_index("s") * NC + lax.axis_index("c")
        base = wid * epw

        def zero_body(j, carry):
            acc[pl.ds(j * 16, 16)] = jnp.zeros((16,), jnp.float32)
            return carry

        lax.fori_loop(0, acc_len // 16, zero_body, 0, unroll=8)

        pltpu.sync_copy(w23_h, wv)
        w2v = [wv[pl.ds(v * 16, 16)] for v in range(4)]
        w3v = [wv[pl.ds(64 + v * 16, 16)] for v in range(4)]
        offv = [jnp.arange(16, dtype=jnp.int32) + v * 16 for v in range(4)]

        def chunk_body(c, carry):
            eb = base + c * CHUNK
            pltpu.sync_copy(eil_h.at[pl.ds(eb, CHUNK)], il)
            pltpu.sync_copy(eia_h.at[pl.ds(eb, CHUNK)], ia)
            pltpu.sync_copy(eax_h.at[pl.ds(eb, CHUNK)], ax)
            pltpu.sync_copy(eay_h.at[pl.ds(eb, CHUNK)], ay)
            pltpu.async_copy(lane2_h.at[il], g, sem).wait()

            def group_body(gi, gcarry):
                axv = ax[pl.ds(gi * 16, 16)]
                ayv = ay[pl.ds(gi * 16, 16)]
                mv = jnp.where(axv * axv + ayv * ayv < RADIUS2,
                               jnp.float32(1.0), jnp.float32(0.0))
                rowb = ia[pl.ds(gi * 16, 16)] * EMBED
                erow = gi * 16
                for e in range(16):
                    ce = jnp.full((16,), e, jnp.int32)
                    axb = axv.at[ce].get(mode="promise_in_bounds")
                    ayb = ayv.at[ce].get(mode="promise_in_bounds")
                    mb = mv.at[ce].get(mode="promise_in_bounds")
                    rb = rowb.at[ce].get(mode="promise_in_bounds")
                    for v in range(4):
                        gv = g[erow + e, pl.ds(v * 16, 16)]
                        m = jnp.maximum(gv + axb * w2v[v] + ayb * w3v[v],
                                        0.0) * mb
                        plsc.addupdate_scatter(acc, [rb + offv[v]], m)
                return gcarry

            lax.fori_loop(0, GROUPS, group_body, 0)
            return carry

        lax.fori_loop(0, nchunk, chunk_body, 0)
        pltpu.sync_copy(acc, out_h.at[wid])

    return k(lane2, eil, eia, eax, eay, w23)


def _tc_post(partials, h0, W_upd, W_dec_xy, b_dec_xy, y60, W_pi_row, b_pi):
    N = h0.shape[0]
    F2 = y60.shape[1]

    def body(p_ref, h0_ref, wupd_ref, wdec_ref, bdec_ref, y60_ref,
             wpi_ref, bpi_ref, yout_ref, pi_ref):
        agg = jnp.sum(p_ref[...], axis=0)
        h = h0_ref[...] + jnp.maximum(
            jnp.dot(agg, wupd_ref[...], preferred_element_type=jnp.float32),
            0.0)
        yout_ref[...] = (y60_ref[...] +
                         jnp.dot(h, wdec_ref[...],
                                 preferred_element_type=jnp.float32) +
                         bdec_ref[...][None, :])
        pi_ref[...] = jnp.sum(h * wpi_ref[...], axis=1) + bpi_ref[...]

    return pl.pallas_call(
        body,
        out_shape=(jax.ShapeDtypeStruct((N, F2), jnp.float32),
                   jax.ShapeDtypeStruct((N,), jnp.float32)),
    )(partials, h0, W_upd, W_dec_xy, b_dec_xy, y60, W_pi_row, b_pi)


def kernel(ego_embed, y_hat, lane_vectors, edge_attr, edge_index_lane,
           edge_index_agent, W_fc, b_fc, W_msg, b_msg, W_upd, W_dec, b_dec,
           W_pi, b_pi):
    N = ego_embed.shape[0]
    E = edge_index_lane.shape[0]
    FUT = y_hat.shape[1]
    L = lane_vectors.shape[0]

    e_pad = _round_up(E, NW * CHUNK)
    pe = e_pad - E
    eil = jnp.concatenate([edge_index_lane, jnp.zeros((pe,), jnp.int32)])
    eia = jnp.concatenate([edge_index_agent, jnp.zeros((pe,), jnp.int32)])
    eax = jnp.concatenate([edge_attr[:, 0],
                           jnp.full((pe,), 1000.0, jnp.float32)])
    eay = jnp.concatenate([edge_attr[:, 1], jnp.zeros((pe,), jnp.float32)])

    Lp = _round_up(L, 128)
    lanes8 = jnp.zeros((Lp, 8), jnp.float32).at[:L, :2].set(lane_vectors)
    Wm8 = jnp.zeros((8, EMBED), jnp.float32).at[:2].set(W_msg[:2])
    lane2, h0 = _tc_pre(lanes8, Wm8, b_msg, ego_embed, W_fc, b_fc)

    w23 = W_msg[2:].reshape(-1)
    partials = _sc_agg(lane2, eil, eia, eax, eay, w23, N)

    W_dec_xy = W_dec.reshape(EMBED, FUT, 4)[:, :, :2].reshape(EMBED, FUT * 2)
    b_dec_xy = b_dec.reshape(FUT, 4)[:, :2].reshape(FUT * 2)
    y60 = y_hat.reshape(N, FUT * 2)
    yout60, pi = _tc_post(partials.reshape(NW, N, EMBED), h0, W_upd,
                          W_dec_xy, b_dec_xy, y60, W_pi.reshape(1, EMBED),
                          b_pi)
    return yout60.reshape(N, FUT, 2), pi


# speedup vs baseline: 4.3072x; 1.4518x over previous
import functools

import jax
import jax.numpy as jnp
from jax import lax
from jax.experimental import pallas as pl
from jax.experimental.pallas import tpu as pltpu
from jax.experimental.pallas import tpu_sc as plsc

EMBED = 64
RADIUS2 = 2500.0
NC, NS = 2, 16
NW = NC * NS
CHUNK = 128
GROUPS = CHUNK // 16


def _round_up(x, m):
    return (x + m - 1) // m * m


def _tc_pre(lanes8, Wm8, b_msg, ego, W_fc, b_fc):
    Lp = lanes8.shape[0]
    N = ego.shape[0]

    def body(lanes_ref, wm_ref, bm_ref, ego_ref, wfc_ref, bfc_ref,
             lane2_ref, h0_ref):
        lane2_ref[...] = (
            jnp.dot(lanes_ref[...], wm_ref[...],
                    preferred_element_type=jnp.float32) + bm_ref[...][None, :])
        h0_ref[...] = (
            jnp.dot(ego_ref[...], wfc_ref[...],
                    preferred_element_type=jnp.float32) + bfc_ref[...][None, :])

    return pl.pallas_call(
        body,
        out_shape=(jax.ShapeDtypeStruct((Lp, EMBED), jnp.float32),
                   jax.ShapeDtypeStruct((N, EMBED), jnp.float32)),
    )(lanes8, Wm8, b_msg, ego, W_fc, b_fc)


def _sc_agg(lane2, eil, eia, eax, eay, w23, nseg):
    acc_len = nseg * EMBED
    e_pad = eil.shape[0] * eil.shape[1]
    epw = e_pad // NW
    nchunk = epw // CHUNK
    mesh = plsc.VectorSubcoreMesh(core_axis_name="c", subcore_axis_name="s",
                                  num_cores=NC, num_subcores=NS)

    @functools.partial(
        pl.kernel,
        out_type=jax.ShapeDtypeStruct((NW, acc_len), jnp.float32),
        mesh=mesh,
        compiler_params=pltpu.CompilerParams(needs_layout_passes=False,
                                             use_tc_tiling_on_sc=False),
        scratch_types=[
            pltpu.VMEM((acc_len,), jnp.float32),
            pltpu.VMEM((2, 1, CHUNK), jnp.int32),
            pltpu.VMEM((2, CHUNK), jnp.int32),
            pltpu.VMEM((2, CHUNK), jnp.float32),
            pltpu.VMEM((2, CHUNK), jnp.float32),
            pltpu.VMEM((2, CHUNK, EMBED), jnp.float32),
            pltpu.VMEM((128,), jnp.float32),
            pltpu.SemaphoreType.DMA,
            pltpu.SemaphoreType.DMA,
            pltpu.SemaphoreType.DMA,
            pltpu.SemaphoreType.DMA,
        ],
    )
    def k(lane2_h, eil_h, eia_h, eax_h, eay_h, w23_h, out_h,
          acc, il2, ia2, ax2, ay2, g2, wv, sem_il, sem_e, sem_g0, sem_g1):
        wid = lax.axis_index("s") * NC + lax.axis_index("c")
        base = wid * epw

        def zero_body(j, carry):
            acc[pl.ds(j * 16, 16)] = jnp.zeros((16,), jnp.float32)
            return carry

        lax.fori_loop(0, acc_len // 16, zero_body, 0, unroll=8)

        pltpu.sync_copy(w23_h, wv)
        w2v = [wv[pl.ds(v * 16, 16)] for v in range(4)]
        w3v = [wv[pl.ds(64 + v * 16, 16)] for v in range(4)]

        def issue_edges(c, p):
            eb = base + c * CHUNK
            row = eb // CHUNK
            pltpu.async_copy(eil_h.at[pl.ds(row, 1)], il2.at[p], sem_il)
            pltpu.async_copy(eia_h.at[pl.ds(eb, CHUNK)], ia2.at[p], sem_e)
            pltpu.async_copy(eax_h.at[pl.ds(eb, CHUNK)], ax2.at[p], sem_e)
            pltpu.async_copy(eay_h.at[pl.ds(eb, CHUNK)], ay2.at[p], sem_e)

        def wait_edges(p):
            pltpu.make_async_copy(eil_h.at[pl.ds(0, 1)], il2.at[p],
                                  sem_il).wait()
            pltpu.make_async_copy(eia_h.at[pl.ds(0, CHUNK)], ia2.at[p],
                                  sem_e).wait()
            pltpu.make_async_copy(eax_h.at[pl.ds(0, CHUNK)], ax2.at[p],
                                  sem_e).wait()
            pltpu.make_async_copy(eay_h.at[pl.ds(0, CHUNK)], ay2.at[p],
                                  sem_e).wait()

        sem_g = (sem_g0, sem_g1)

        def issue_gather(p):
            pltpu.async_copy(lane2_h.at[il2.at[p, 0]], g2.at[p], sem_g[p])

        def wait_gather(p):
            pltpu.make_async_copy(lane2_h.at[il2.at[p, 0]], g2.at[p],
                                  sem_g[p]).wait()

        issue_edges(0, 0)
        wait_edges(0)
        issue_gather(0)
        issue_edges(1, 1)

        def half_body(c, p):
            q = 1 - p

            @pl.when(c + 1 < nchunk)
            def _():
                wait_edges(q)
                issue_gather(q)

            wait_gather(p)

            def group_body(gi, gcarry):
                iav = ia2[p, pl.ds(gi * 16, 16)]
                axv = ax2[p, pl.ds(gi * 16, 16)]
                ayv = ay2[p, pl.ds(gi * 16, 16)]
                mv = jnp.where(axv * axv + ayv * ayv < RADIUS2,
                               jnp.float32(1.0), jnp.float32(0.0))
                rowb = iav * EMBED
                erow = gi * 16
                for e in range(16):
                    ce = jnp.full((16,), e, jnp.int32)
                    axb = axv.at[ce].get(mode="promise_in_bounds")
                    ayb = ayv.at[ce].get(mode="promise_in_bounds")
                    mb = mv.at[ce].get(mode="promise_in_bounds")
                    rb = rowb.at[ce].get(mode="promise_in_bounds")
                    for v in range(4):
                        gv = g2[p, erow + e, pl.ds(v * 16, 16)]
                        m = jnp.maximum(gv + axb * w2v[v] + ayb * w3v[v],
                                        0.0) * mb
                        idx = rb + (jnp.arange(16, dtype=jnp.int32) + v * 16)
                        plsc.addupdate_scatter(acc, [idx], m)
                return gcarry

            lax.fori_loop(0, GROUPS, group_body, 0)

            @pl.when(c + 2 < nchunk)
            def _():
                issue_edges(c + 2, p)

        def pair_body(m, carry):
            half_body(2 * m, 0)
            half_body(2 * m + 1, 1)
            return carry

        lax.fori_loop(0, nchunk // 2, pair_body, 0)
        pltpu.sync_copy(acc, out_h.at[wid])

    return k(lane2, eil, eia, eax, eay, w23)


def _tc_post(partials, h0, W_upd, W_dec_xy, b_dec_xy, y60, W_pi_row, b_pi):
    N = h0.shape[0]
    F2 = y60.shape[1]

    def body(p_ref, h0_ref, wupd_ref, wdec_ref, bdec_ref, y60_ref,
             wpi_ref, bpi_ref, yout_ref, pi_ref):
        agg = jnp.sum(p_ref[...], axis=0)
        h = h0_ref[...] + jnp.maximum(
            jnp.dot(agg, wupd_ref[...], preferred_element_type=jnp.float32),
            0.0)
        yout_ref[...] = (y60_ref[...] +
                         jnp.dot(h, wdec_ref[...],
                                 preferred_element_type=jnp.float32) +
                         bdec_ref[...][None, :])
        pi_ref[...] = jnp.sum(h * wpi_ref[...], axis=1) + bpi_ref[...]

    return pl.pallas_call(
        body,
        out_shape=(jax.ShapeDtypeStruct((N, F2), jnp.float32),
                   jax.ShapeDtypeStruct((N,), jnp.float32)),
    )(partials, h0, W_upd, W_dec_xy, b_dec_xy, y60, W_pi_row, b_pi)


def kernel(ego_embed, y_hat, lane_vectors, edge_attr, edge_index_lane,
           edge_index_agent, W_fc, b_fc, W_msg, b_msg, W_upd, W_dec, b_dec,
           W_pi, b_pi):
    N = ego_embed.shape[0]
    E = edge_index_lane.shape[0]
    FUT = y_hat.shape[1]
    L = lane_vectors.shape[0]

    e_pad = _round_up(E, NW * CHUNK * 2)
    pe = e_pad - E
    eil = jnp.concatenate([edge_index_lane, jnp.zeros((pe,), jnp.int32)])
    eil = eil.reshape(e_pad // CHUNK, CHUNK)
    eia = jnp.concatenate([edge_index_agent, jnp.zeros((pe,), jnp.int32)])
    eax = jnp.concatenate([edge_attr[:, 0],
                           jnp.full((pe,), 1000.0, jnp.float32)])
    eay = jnp.concatenate([edge_attr[:, 1], jnp.zeros((pe,), jnp.float32)])

    Lp = _round_up(L, 128)
    lanes8 = jnp.zeros((Lp, 8), jnp.float32).at[:L, :2].set(lane_vectors)
    Wm8 = jnp.zeros((8, EMBED), jnp.float32).at[:2].set(W_msg[:2])
    lane2, h0 = _tc_pre(lanes8, Wm8, b_msg, ego_embed, W_fc, b_fc)

    w23 = W_msg[2:].reshape(-1)
    partials = _sc_agg(lane2, eil, eia, eax, eay, w23, N)

    W_dec_xy = W_dec.reshape(EMBED, FUT, 4)[:, :, :2].reshape(EMBED, FUT * 2)
    b_dec_xy = b_dec.reshape(FUT, 4)[:, :2].reshape(FUT * 2)
    y60 = y_hat.reshape(N, FUT * 2)
    yout60, pi = _tc_post(partials.reshape(NW, N, EMBED), h0, W_upd,
                          W_dec_xy, b_dec_xy, y60, W_pi.reshape(1, EMBED),
                          b_pi)
    return yout60.reshape(N, FUT, 2), pi


# speedup vs baseline: 8.0463x; 1.8681x over previous
import functools

import jax
import jax.numpy as jnp
from jax import lax
from jax.experimental import pallas as pl
from jax.experimental.pallas import tpu as pltpu
from jax.experimental.pallas import tpu_sc as plsc

EMBED = 64
RADIUS2 = 2500.0
NC, NS = 2, 16
NW = NC * NS
CHUNK = 128
GROUPS = CHUNK // 16


def _round_up(x, m):
    return (x + m - 1) // m * m


def _tc_pre(lanes8, Wm8, b_msg, ego, W_fc, b_fc):
    Lp = lanes8.shape[0]
    N = ego.shape[0]

    def body(lanes_ref, wm_ref, bm_ref, ego_ref, wfc_ref, bfc_ref,
             lane2_ref, h0_ref):
        lane2_ref[...] = (
            jnp.dot(lanes_ref[...], wm_ref[...],
                    preferred_element_type=jnp.float32) + bm_ref[...][None, :])
        h0_ref[...] = (
            jnp.dot(ego_ref[...], wfc_ref[...],
                    preferred_element_type=jnp.float32) + bfc_ref[...][None, :])

    return pl.pallas_call(
        body,
        out_shape=(jax.ShapeDtypeStruct((Lp, EMBED), jnp.float32),
                   jax.ShapeDtypeStruct((N, EMBED), jnp.float32)),
    )(lanes8, Wm8, b_msg, ego, W_fc, b_fc)


def _sc_agg(lane2, eil, eia, eax, eay, w23, nseg):
    acc_len = nseg * EMBED
    e_pad = eil.shape[0] * eil.shape[1]
    epw = e_pad // NW
    nchunk = epw // CHUNK
    mesh = plsc.VectorSubcoreMesh(core_axis_name="c", subcore_axis_name="s",
                                  num_cores=NC, num_subcores=NS)

    @functools.partial(
        pl.kernel,
        out_type=jax.ShapeDtypeStruct((NW, acc_len), jnp.float32),
        mesh=mesh,
        compiler_params=pltpu.CompilerParams(needs_layout_passes=False,
                                             use_tc_tiling_on_sc=False),
        scratch_types=[
            pltpu.VMEM((acc_len,), jnp.float32),
            pltpu.VMEM((2, 1, CHUNK), jnp.int32),
            pltpu.VMEM((2, CHUNK), jnp.int32),
            pltpu.VMEM((2, CHUNK), jnp.float32),
            pltpu.VMEM((2, CHUNK), jnp.float32),
            pltpu.VMEM((2, CHUNK, EMBED), jnp.float32),
            pltpu.VMEM((128,), jnp.float32),
            pltpu.SemaphoreType.DMA,
            pltpu.SemaphoreType.DMA,
            pltpu.SemaphoreType.DMA,
            pltpu.SemaphoreType.DMA,
        ],
    )
    def k(lane2_h, eil_h, eia_h, eax_h, eay_h, w23_h, out_h,
          acc, il2, ia2, ax2, ay2, g2, wv, sem_il, sem_e, sem_g0, sem_g1):
        wid = lax.axis_index("s") * NC + lax.axis_index("c")
        base = wid * epw

        @plsc.parallel_loop(0, acc_len // 16, unroll=8)
        def _(j):
            acc[pl.ds(j * 16, 16)] = jnp.zeros((16,), jnp.float32)

        pltpu.sync_copy(w23_h, wv)
        w2v = [wv[pl.ds(v * 16, 16)] for v in range(4)]
        w3v = [wv[pl.ds(64 + v * 16, 16)] for v in range(4)]

        def issue_edges(c, p):
            eb = base + c * CHUNK
            row = eb // CHUNK
            pltpu.async_copy(eil_h.at[pl.ds(row, 1)], il2.at[p], sem_il)
            pltpu.async_copy(eia_h.at[pl.ds(eb, CHUNK)], ia2.at[p], sem_e)
            pltpu.async_copy(eax_h.at[pl.ds(eb, CHUNK)], ax2.at[p], sem_e)
            pltpu.async_copy(eay_h.at[pl.ds(eb, CHUNK)], ay2.at[p], sem_e)

        def wait_edges(p):
            pltpu.make_async_copy(eil_h.at[pl.ds(0, 1)], il2.at[p],
                                  sem_il).wait()
            pltpu.make_async_copy(eia_h.at[pl.ds(0, CHUNK)], ia2.at[p],
                                  sem_e).wait()
            pltpu.make_async_copy(eax_h.at[pl.ds(0, CHUNK)], ax2.at[p],
                                  sem_e).wait()
            pltpu.make_async_copy(eay_h.at[pl.ds(0, CHUNK)], ay2.at[p],
                                  sem_e).wait()

        sem_g = (sem_g0, sem_g1)

        def issue_gather(p):
            pltpu.async_copy(lane2_h.at[il2.at[p, 0]], g2.at[p], sem_g[p])

        def wait_gather(p):
            pltpu.make_async_copy(lane2_h.at[il2.at[p, 0]], g2.at[p],
                                  sem_g[p]).wait()

        issue_edges(0, 0)
        wait_edges(0)
        issue_gather(0)
        issue_edges(1, 1)

        def half_body(c, p):
            q = 1 - p

            @pl.when(c + 1 < nchunk)
            def _():
                wait_edges(q)
                issue_gather(q)

            wait_gather(p)

            def group_body(gi, gcarry):
                iav = ia2[p, pl.ds(gi * 16, 16)]
                axv = ax2[p, pl.ds(gi * 16, 16)]
                ayv = ay2[p, pl.ds(gi * 16, 16)]
                mv = jnp.where(axv * axv + ayv * ayv < RADIUS2,
                               jnp.float32(1.0), jnp.float32(0.0))
                rowb = iav * EMBED
                erow = gi * 16

                @plsc.parallel_loop(0, 16, unroll=4)
                def _(e):
                    ce = jnp.full((16,), 0, jnp.int32) + e
                    axb = axv.at[ce].get(mode="promise_in_bounds")
                    ayb = ayv.at[ce].get(mode="promise_in_bounds")
                    mb = mv.at[ce].get(mode="promise_in_bounds")
                    rb = rowb.at[ce].get(mode="promise_in_bounds")
                    for v in range(4):
                        gv = g2[p, erow + e, pl.ds(v * 16, 16)]
                        m = jnp.maximum(gv + axb * w2v[v] + ayb * w3v[v],
                                        0.0) * mb
                        idx = rb + (jnp.arange(16, dtype=jnp.int32) + v * 16)
                        plsc.addupdate_scatter(acc, [idx], m)

                return gcarry

            lax.fori_loop(0, GROUPS, group_body, 0)

            @pl.when(c + 2 < nchunk)
            def _():
                issue_edges(c + 2, p)

        def pair_body(m, carry):
            half_body(2 * m, 0)
            half_body(2 * m + 1, 1)
            return carry

        lax.fori_loop(0, nchunk // 2, pair_body, 0)
        pltpu.sync_copy(acc, out_h.at[wid])

    return k(lane2, eil, eia, eax, eay, w23)


def _tc_post(partials, h0, W_upd, W_dec_xy, b_dec_xy, y60, W_pi_row, b_pi):
    N = h0.shape[0]
    F2 = y60.shape[1]

    def body(p_ref, h0_ref, wupd_ref, wdec_ref, bdec_ref, y60_ref,
             wpi_ref, bpi_ref, yout_ref, pi_ref):
        agg = jnp.sum(p_ref[...], axis=0)
        h = h0_ref[...] + jnp.maximum(
            jnp.dot(agg, wupd_ref[...], preferred_element_type=jnp.float32),
            0.0)
        yout_ref[...] = (y60_ref[...] +
                         jnp.dot(h, wdec_ref[...],
                                 preferred_element_type=jnp.float32) +
                         bdec_ref[...][None, :])
        pi_ref[...] = jnp.sum(h * wpi_ref[...], axis=1) + bpi_ref[...]

    return pl.pallas_call(
        body,
        out_shape=(jax.ShapeDtypeStruct((N, F2), jnp.float32),
                   jax.ShapeDtypeStruct((N,), jnp.float32)),
    )(partials, h0, W_upd, W_dec_xy, b_dec_xy, y60, W_pi_row, b_pi)


def kernel(ego_embed, y_hat, lane_vectors, edge_attr, edge_index_lane,
           edge_index_agent, W_fc, b_fc, W_msg, b_msg, W_upd, W_dec, b_dec,
           W_pi, b_pi):
    N = ego_embed.shape[0]
    E = edge_index_lane.shape[0]
    FUT = y_hat.shape[1]
    L = lane_vectors.shape[0]

    e_pad = _round_up(E, NW * CHUNK * 2)
    pe = e_pad - E
    eil = jnp.concatenate([edge_index_lane, jnp.zeros((pe,), jnp.int32)])
    eil = eil.reshape(e_pad // CHUNK, CHUNK)
    eia = jnp.concatenate([edge_index_agent, jnp.zeros((pe,), jnp.int32)])
    eax = jnp.concatenate([edge_attr[:, 0],
                           jnp.full((pe,), 1000.0, jnp.float32)])
    eay = jnp.concatenate([edge_attr[:, 1], jnp.zeros((pe,), jnp.float32)])

    Lp = _round_up(L, 128)
    lanes8 = jnp.zeros((Lp, 8), jnp.float32).at[:L, :2].set(lane_vectors)
    Wm8 = jnp.zeros((8, EMBED), jnp.float32).at[:2].set(W_msg[:2])
    lane2, h0 = _tc_pre(lanes8, Wm8, b_msg, ego_embed, W_fc, b_fc)

    w23 = W_msg[2:].reshape(-1)
    partials = _sc_agg(lane2, eil, eia, eax, eay, w23, N)

    W_dec_xy = W_dec.reshape(EMBED, FUT, 4)[:, :, :2].reshape(EMBED, FUT * 2)
    b_dec_xy = b_dec.reshape(FUT, 4)[:, :2].reshape(FUT * 2)
    y60 = y_hat.reshape(N, FUT * 2)
    yout60, pi = _tc_post(partials.reshape(NW, N, EMBED), h0, W_upd,
                          W_dec_xy, b_dec_xy, y60, W_pi.reshape(1, EMBED),
                          b_pi)
    return yout60.reshape(N, FUT, 2), pi


# speedup vs baseline: 9.6082x; 1.1941x over previous
import functools

import jax
import jax.numpy as jnp
from jax import lax
from jax.experimental import pallas as pl
from jax.experimental.pallas import tpu as pltpu
from jax.experimental.pallas import tpu_sc as plsc

EMBED = 64
RADIUS2 = 2500.0
NC, NS = 2, 16
NW = NC * NS
CHUNK = 128
GROUPS = CHUNK // 16


def _round_up(x, m):
    return (x + m - 1) // m * m


def _tc_pre(lanes8, Wm8, b_msg, ego, W_fc, b_fc):
    Lp = lanes8.shape[0]
    N = ego.shape[0]

    def body(lanes_ref, wm_ref, bm_ref, ego_ref, wfc_ref, bfc_ref,
             lane2_ref, h0_ref):
        lane2_ref[...] = (
            jnp.dot(lanes_ref[...], wm_ref[...],
                    preferred_element_type=jnp.float32) + bm_ref[...][None, :])
        h0_ref[...] = (
            jnp.dot(ego_ref[...], wfc_ref[...],
                    preferred_element_type=jnp.float32) + bfc_ref[...][None, :])

    return pl.pallas_call(
        body,
        out_shape=(jax.ShapeDtypeStruct((Lp, EMBED), jnp.float32),
                   jax.ShapeDtypeStruct((N, EMBED), jnp.float32)),
    )(lanes8, Wm8, b_msg, ego, W_fc, b_fc)


def _sc_agg(lane2, eil, eia, eax, eay, w23, nseg):
    acc_len = nseg * EMBED
    e_pad = eil.shape[0] * eil.shape[1]
    epw = e_pad // NW
    nchunk = epw // CHUNK
    mesh = plsc.VectorSubcoreMesh(core_axis_name="c", subcore_axis_name="s",
                                  num_cores=NC, num_subcores=NS)

    @functools.partial(
        pl.kernel,
        out_type=jax.ShapeDtypeStruct((NC, nseg, EMBED), jnp.float32),
        mesh=mesh,
        compiler_params=pltpu.CompilerParams(needs_layout_passes=False,
                                             use_tc_tiling_on_sc=False),
        scratch_types=[
            pltpu.VMEM_SHARED((nseg, EMBED), jnp.float32),
            pltpu.VMEM((nseg // NS, EMBED), jnp.float32),
            pltpu.VMEM((4, 1, CHUNK), jnp.int32),
            pltpu.VMEM((4, CHUNK), jnp.int32),
            pltpu.VMEM((4, CHUNK), jnp.float32),
            pltpu.VMEM((4, CHUNK), jnp.float32),
            pltpu.VMEM((2, CHUNK, EMBED), jnp.float32),
            pltpu.VMEM((2, CHUNK, EMBED), jnp.float32),
            pltpu.VMEM((128,), jnp.float32),
            pltpu.SemaphoreType.DMA,
            pltpu.SemaphoreType.DMA,
            pltpu.SemaphoreType.DMA,
            pltpu.SemaphoreType.DMA,
            pltpu.SemaphoreType.DMA,
            pltpu.SemaphoreType.DMA,
        ],
    )
    def k(lane2_h, eil_h, eia_h, eax_h, eay_h, w23_h, out_h,
          acc, zbuf, il4, ia4, ax4, ay4, g2, msg2, wv,
          sem_il, sem_e, sem_g0, sem_g1, sem_s0, sem_s1):
        cid = lax.axis_index("c")
        sid = lax.axis_index("s")
        wid = sid * NC + cid
        base = wid * epw
        zrows = nseg // NS

        @plsc.parallel_loop(0, zrows, unroll=2)
        def _(j):
            for v in range(4):
                zbuf[j, pl.ds(v * 16, 16)] = jnp.zeros((16,), jnp.float32)

        pltpu.sync_copy(zbuf, acc.at[pl.ds(sid * zrows, zrows)])
        plsc.subcore_barrier()

        pltpu.sync_copy(w23_h, wv)
        w2v = [wv[pl.ds(v * 16, 16)] for v in range(4)]
        w3v = [wv[pl.ds(64 + v * 16, 16)] for v in range(4)]

        sem_g = (sem_g0, sem_g1)
        sem_s = (sem_s0, sem_s1)

        def issue_edges(c, p4):
            eb = base + c * CHUNK
            row = eb // CHUNK
            pltpu.async_copy(eil_h.at[pl.ds(row, 1)], il4.at[p4], sem_il)
            pltpu.async_copy(eia_h.at[pl.ds(eb, CHUNK)], ia4.at[p4], sem_e)
            pltpu.async_copy(eax_h.at[pl.ds(eb, CHUNK)], ax4.at[p4], sem_e)
            pltpu.async_copy(eay_h.at[pl.ds(eb, CHUNK)], ay4.at[p4], sem_e)

        def wait_edges(p4):
            pltpu.make_async_copy(eil_h.at[pl.ds(0, 1)], il4.at[p4],
                                  sem_il).wait()
            pltpu.make_async_copy(eia_h.at[pl.ds(0, CHUNK)], ia4.at[p4],
                                  sem_e).wait()
            pltpu.make_async_copy(eax_h.at[pl.ds(0, CHUNK)], ax4.at[p4],
                                  sem_e).wait()
            pltpu.make_async_copy(eay_h.at[pl.ds(0, CHUNK)], ay4.at[p4],
                                  sem_e).wait()

        def issue_gather(p2, p4):
            pltpu.async_copy(lane2_h.at[il4.at[p4, 0]], g2.at[p2],
                             sem_g[p2])

        def wait_gather(p2, p4):
            pltpu.make_async_copy(lane2_h.at[il4.at[p4, 0]], g2.at[p2],
                                  sem_g[p2]).wait()

        def issue_scatter(p2, p4):
            pltpu.async_copy(msg2.at[p2], acc.at[ia4.at[p4]], sem_s[p2],
                             add=True)

        def wait_scatter(p2, p4):
            pltpu.make_async_copy(msg2.at[p2], acc.at[ia4.at[p4]],
                                  sem_s[p2]).wait()

        issue_edges(0, 0)
        wait_edges(0)
        issue_gather(0, 0)
        issue_edges(1, 1)

        def quarter_body(c, p2, p4):
            @pl.when(c + 1 < nchunk)
            def _():
                wait_edges((p4 + 1) % 4)
                issue_gather(1 - p2, (p4 + 1) % 4)

            wait_gather(p2, p4)

            @pl.when(c >= 2)
            def _():
                wait_scatter(p2, p4)

            def group_body(gi, gcarry):
                axv = ax4[p4, pl.ds(gi * 16, 16)]
                ayv = ay4[p4, pl.ds(gi * 16, 16)]
                mv = jnp.where(axv * axv + ayv * ayv < RADIUS2,
                               jnp.float32(1.0), jnp.float32(0.0))
                erow = gi * 16

                @plsc.parallel_loop(0, 16, unroll=4)
                def _(e):
                    ce = jnp.full((16,), 0, jnp.int32) + e
                    axb = axv.at[ce].get(mode="promise_in_bounds")
                    ayb = ayv.at[ce].get(mode="promise_in_bounds")
                    mb = mv.at[ce].get(mode="promise_in_bounds")
                    for v in range(4):
                        gv = g2[p2, erow + e, pl.ds(v * 16, 16)]
                        m = jnp.maximum(gv + axb * w2v[v] + ayb * w3v[v],
                                        0.0) * mb
                        msg2[p2, erow + e, pl.ds(v * 16, 16)] = m

                return gcarry

            lax.fori_loop(0, GROUPS, group_body, 0)
            issue_scatter(p2, p4)

            @pl.when(c + 2 < nchunk)
            def _():
                issue_edges(c + 2, (p4 + 2) % 4)

        def quad_body(m, carry):
            c0 = 4 * m
            quarter_body(c0, 0, 0)
            quarter_body(c0 + 1, 1, 1)
            quarter_body(c0 + 2, 0, 2)
            quarter_body(c0 + 3, 1, 3)
            return carry

        lax.fori_loop(0, nchunk // 4, quad_body, 0)
        wait_scatter(0, 2)
        wait_scatter(1, 3)
        plsc.subcore_barrier()

        @pl.when(sid == 0)
        def _():
            pltpu.sync_copy(acc, out_h.at[cid])

    return k(lane2, eil, eia, eax, eay, w23)


def _tc_post(partials, h0, W_upd, W_dec_xy, b_dec_xy, y60, W_pi_row, b_pi):
    N = h0.shape[0]
    F2 = y60.shape[1]

    def body(p_ref, h0_ref, wupd_ref, wdec_ref, bdec_ref, y60_ref,
             wpi_ref, bpi_ref, yout_ref, pi_ref):
        agg = jnp.sum(p_ref[...], axis=0)
        h = h0_ref[...] + jnp.maximum(
            jnp.dot(agg, wupd_ref[...], preferred_element_type=jnp.float32),
            0.0)
        yout_ref[...] = (y60_ref[...] +
                         jnp.dot(h, wdec_ref[...],
                                 preferred_element_type=jnp.float32) +
                         bdec_ref[...][None, :])
        pi_ref[...] = jnp.sum(h * wpi_ref[...], axis=1) + bpi_ref[...]

    return pl.pallas_call(
        body,
        out_shape=(jax.ShapeDtypeStruct((N, F2), jnp.float32),
                   jax.ShapeDtypeStruct((N,), jnp.float32)),
    )(partials, h0, W_upd, W_dec_xy, b_dec_xy, y60, W_pi_row, b_pi)


def kernel(ego_embed, y_hat, lane_vectors, edge_attr, edge_index_lane,
           edge_index_agent, W_fc, b_fc, W_msg, b_msg, W_upd, W_dec, b_dec,
           W_pi, b_pi):
    N = ego_embed.shape[0]
    E = edge_index_lane.shape[0]
    FUT = y_hat.shape[1]
    L = lane_vectors.shape[0]

    e_pad = _round_up(E, NW * CHUNK * 4)
    pe = e_pad - E
    eil = jnp.concatenate([edge_index_lane, jnp.zeros((pe,), jnp.int32)])
    eil = eil.reshape(e_pad // CHUNK, CHUNK)
    eia = jnp.concatenate([edge_index_agent, jnp.zeros((pe,), jnp.int32)])
    eax = jnp.concatenate([edge_attr[:, 0],
                           jnp.full((pe,), 1000.0, jnp.float32)])
    eay = jnp.concatenate([edge_attr[:, 1], jnp.zeros((pe,), jnp.float32)])

    Lp = _round_up(L, 128)
    lanes8 = jnp.zeros((Lp, 8), jnp.float32).at[:L, :2].set(lane_vectors)
    Wm8 = jnp.zeros((8, EMBED), jnp.float32).at[:2].set(W_msg[:2])
    lane2, h0 = _tc_pre(lanes8, Wm8, b_msg, ego_embed, W_fc, b_fc)

    w23 = W_msg[2:].reshape(-1)
    partials = _sc_agg(lane2, eil, eia, eax, eay, w23, N)

    W_dec_xy = W_dec.reshape(EMBED, FUT, 4)[:, :, :2].reshape(EMBED, FUT * 2)
    b_dec_xy = b_dec.reshape(FUT, 4)[:, :2].reshape(FUT * 2)
    y60 = y_hat.reshape(N, FUT * 2)
    yout60, pi = _tc_post(partials, h0, W_upd,
                          W_dec_xy, b_dec_xy, y60, W_pi.reshape(1, EMBED),
                          b_pi)
    return yout60.reshape(N, FUT, 2), pi


# speedup vs baseline: 10.2101x; 1.0626x over previous
import functools

import jax
import jax.numpy as jnp
from jax import lax
from jax.experimental import pallas as pl
from jax.experimental.pallas import tpu as pltpu
from jax.experimental.pallas import tpu_sc as plsc

EMBED = 64
RADIUS2 = 2500.0
NC, NS = 2, 16
NW = NC * NS
CHUNK = 128
GROUPS = CHUNK // 16


def _round_up(x, m):
    return (x + m - 1) // m * m


def _tc_pre(lanes8, Wm8, b_msg, ego, W_fc, b_fc):
    Lp = lanes8.shape[0]
    N = ego.shape[0]

    def body(lanes_ref, wm_ref, bm_ref, ego_ref, wfc_ref, bfc_ref,
             lane2_ref, h0_ref):
        lane2_ref[...] = (
            jnp.dot(lanes_ref[...], wm_ref[...],
                    preferred_element_type=jnp.float32) + bm_ref[...][None, :])
        h0_ref[...] = (
            jnp.dot(ego_ref[...], wfc_ref[...],
                    preferred_element_type=jnp.float32) + bfc_ref[...][None, :])

    return pl.pallas_call(
        body,
        out_shape=(jax.ShapeDtypeStruct((Lp, EMBED), jnp.float32),
                   jax.ShapeDtypeStruct((N, EMBED), jnp.float32)),
    )(lanes8, Wm8, b_msg, ego, W_fc, b_fc)


def _sc_agg(lane2, eil, eia, eax, eay, w23, nseg):
    acc_len = nseg * EMBED
    e_pad = eil.shape[0] * eil.shape[1]
    epw = e_pad // NW
    nchunk = epw // CHUNK
    mesh = plsc.VectorSubcoreMesh(core_axis_name="c", subcore_axis_name="s",
                                  num_cores=NC, num_subcores=NS)

    @functools.partial(
        pl.kernel,
        out_type=jax.ShapeDtypeStruct((NC, nseg, EMBED), jnp.float32),
        mesh=mesh,
        compiler_params=pltpu.CompilerParams(needs_layout_passes=False,
                                             use_tc_tiling_on_sc=False),
        scratch_types=[
            pltpu.VMEM_SHARED((nseg + NS, EMBED), jnp.float32),
            pltpu.VMEM((nseg // NS + 1, EMBED), jnp.float32),
            pltpu.VMEM((2, CHUNK), jnp.int32),
            pltpu.VMEM((4, 1, CHUNK), jnp.int32),
            pltpu.VMEM((4, CHUNK), jnp.int32),
            pltpu.VMEM((4, CHUNK), jnp.float32),
            pltpu.VMEM((4, CHUNK), jnp.float32),
            pltpu.VMEM((2, CHUNK, EMBED), jnp.float32),
            pltpu.VMEM((2, CHUNK, EMBED), jnp.float32),
            pltpu.VMEM((128,), jnp.float32),
            pltpu.SemaphoreType.DMA,
            pltpu.SemaphoreType.DMA,
            pltpu.SemaphoreType.DMA,
            pltpu.SemaphoreType.DMA,
            pltpu.SemaphoreType.DMA,
            pltpu.SemaphoreType.DMA,
        ],
    )
    def k(lane2_h, eil_h, eia_h, eax_h, eay_h, w23_h, out_h,
          acc, zbuf, idxb, il4, ia4, ax4, ay4, g2, msg2, wv,
          sem_il, sem_e, sem_g0, sem_g1, sem_s0, sem_s1):
        cid = lax.axis_index("c")
        sid = lax.axis_index("s")
        wid = sid * NC + cid
        base = wid * epw
        zrows = nseg // NS

        @plsc.parallel_loop(0, zrows + 1, unroll=2)
        def _(j):
            for v in range(4):
                zbuf[j, pl.ds(v * 16, 16)] = jnp.zeros((16,), jnp.float32)

        pltpu.sync_copy(zbuf.at[pl.ds(0, zrows)],
                        acc.at[pl.ds(sid * zrows, zrows)])
        pltpu.sync_copy(zbuf.at[pl.ds(zrows, 1)],
                        acc.at[pl.ds(nseg + sid, 1)])
        plsc.subcore_barrier()

        pltpu.sync_copy(w23_h, wv)
        w2v = [wv[pl.ds(v * 16, 16)] for v in range(4)]
        w3v = [wv[pl.ds(64 + v * 16, 16)] for v in range(4)]

        sem_g = (sem_g0, sem_g1)
        sem_s = (sem_s0, sem_s1)

        def issue_edges(c, p4):
            eb = base + c * CHUNK
            row = eb // CHUNK
            pltpu.async_copy(eil_h.at[pl.ds(row, 1)], il4.at[p4], sem_il)
            pltpu.async_copy(eia_h.at[pl.ds(eb, CHUNK)], ia4.at[p4], sem_e)
            pltpu.async_copy(eax_h.at[pl.ds(eb, CHUNK)], ax4.at[p4], sem_e)
            pltpu.async_copy(eay_h.at[pl.ds(eb, CHUNK)], ay4.at[p4], sem_e)

        def wait_edges(p4):
            pltpu.make_async_copy(eil_h.at[pl.ds(0, 1)], il4.at[p4],
                                  sem_il).wait()
            pltpu.make_async_copy(eia_h.at[pl.ds(0, CHUNK)], ia4.at[p4],
                                  sem_e).wait()
            pltpu.make_async_copy(eax_h.at[pl.ds(0, CHUNK)], ax4.at[p4],
                                  sem_e).wait()
            pltpu.make_async_copy(eay_h.at[pl.ds(0, CHUNK)], ay4.at[p4],
                                  sem_e).wait()

        def issue_gather(p2, p4):
            pltpu.async_copy(lane2_h.at[il4.at[p4, 0]], g2.at[p2],
                             sem_g[p2])

        def wait_gather(p2, p4):
            pltpu.make_async_copy(lane2_h.at[il4.at[p4, 0]], g2.at[p2],
                                  sem_g[p2]).wait()

        def issue_scatter(p2, p4):
            pltpu.async_copy(msg2.at[p2], acc.at[idxb.at[p2]], sem_s[p2],
                             add=True)

        def wait_scatter(p2, p4):
            pltpu.make_async_copy(msg2.at[p2], acc.at[idxb.at[p2]],
                                  sem_s[p2]).wait()

        issue_edges(0, 0)
        wait_edges(0)
        issue_gather(0, 0)
        issue_edges(1, 1)

        def quarter_body(c, p2, p4):
            @pl.when(c + 1 < nchunk)
            def _():
                wait_edges((p4 + 1) % 4)
                issue_gather(1 - p2, (p4 + 1) % 4)

            wait_gather(p2, p4)

            @pl.when(c >= 2)
            def _():
                wait_scatter(p2, p4)

            dummy = jnp.full((16,), nseg, jnp.int32) + sid

            def group_body(gi, gcarry):
                axv = ax4[p4, pl.ds(gi * 16, 16)]
                ayv = ay4[p4, pl.ds(gi * 16, 16)]
                iav = ia4[p4, pl.ds(gi * 16, 16)]
                idxb[p2, pl.ds(gi * 16, 16)] = jnp.where(
                    axv * axv + ayv * ayv < RADIUS2, iav, dummy)
                erow = gi * 16

                @plsc.parallel_loop(0, 16, unroll=4)
                def _(e):
                    ce = jnp.full((16,), 0, jnp.int32) + e
                    axb = axv.at[ce].get(mode="promise_in_bounds")
                    ayb = ayv.at[ce].get(mode="promise_in_bounds")
                    for v in range(4):
                        gv = g2[p2, erow + e, pl.ds(v * 16, 16)]
                        m = jnp.maximum(gv + axb * w2v[v] + ayb * w3v[v],
                                        0.0)
                        msg2[p2, erow + e, pl.ds(v * 16, 16)] = m

                return gcarry

            lax.fori_loop(0, GROUPS, group_body, 0)
            issue_scatter(p2, p4)

            @pl.when(c + 2 < nchunk)
            def _():
                issue_edges(c + 2, (p4 + 2) % 4)

        def quad_body(m, carry):
            c0 = 4 * m
            quarter_body(c0, 0, 0)
            quarter_body(c0 + 1, 1, 1)
            quarter_body(c0 + 2, 0, 2)
            quarter_body(c0 + 3, 1, 3)
            return carry

        lax.fori_loop(0, nchunk // 4, quad_body, 0)
        wait_scatter(0, 2)
        wait_scatter(1, 3)
        plsc.subcore_barrier()

        @pl.when(sid == 0)
        def _():
            pltpu.sync_copy(acc.at[pl.ds(0, nseg)], out_h.at[cid])

    return k(lane2, eil, eia, eax, eay, w23)


def _tc_post(partials, h0, W_upd, W_dec_xy, b_dec_xy, y60, W_pi_row, b_pi):
    N = h0.shape[0]
    F2 = y60.shape[1]

    def body(p_ref, h0_ref, wupd_ref, wdec_ref, bdec_ref, y60_ref,
             wpi_ref, bpi_ref, yout_ref, pi_ref):
        agg = jnp.sum(p_ref[...], axis=0)
        h = h0_ref[...] + jnp.maximum(
            jnp.dot(agg, wupd_ref[...], preferred_element_type=jnp.float32),
            0.0)
        yout_ref[...] = (y60_ref[...] +
                         jnp.dot(h, wdec_ref[...],
                                 preferred_element_type=jnp.float32) +
                         bdec_ref[...][None, :])
        pi_ref[...] = jnp.sum(h * wpi_ref[...], axis=1) + bpi_ref[...]

    return pl.pallas_call(
        body,
        out_shape=(jax.ShapeDtypeStruct((N, F2), jnp.float32),
                   jax.ShapeDtypeStruct((N,), jnp.float32)),
    )(partials, h0, W_upd, W_dec_xy, b_dec_xy, y60, W_pi_row, b_pi)


def kernel(ego_embed, y_hat, lane_vectors, edge_attr, edge_index_lane,
           edge_index_agent, W_fc, b_fc, W_msg, b_msg, W_upd, W_dec, b_dec,
           W_pi, b_pi):
    N = ego_embed.shape[0]
    E = edge_index_lane.shape[0]
    FUT = y_hat.shape[1]
    L = lane_vectors.shape[0]

    e_pad = _round_up(E, NW * CHUNK * 4)
    pe = e_pad - E
    eil = jnp.concatenate([edge_index_lane, jnp.zeros((pe,), jnp.int32)])
    eil = eil.reshape(e_pad // CHUNK, CHUNK)
    eia = jnp.concatenate([edge_index_agent, jnp.zeros((pe,), jnp.int32)])
    eax = jnp.concatenate([edge_attr[:, 0],
                           jnp.full((pe,), 1000.0, jnp.float32)])
    eay = jnp.concatenate([edge_attr[:, 1], jnp.zeros((pe,), jnp.float32)])

    Lp = _round_up(L, 128)
    lanes8 = jnp.zeros((Lp, 8), jnp.float32).at[:L, :2].set(lane_vectors)
    Wm8 = jnp.zeros((8, EMBED), jnp.float32).at[:2].set(W_msg[:2])
    lane2, h0 = _tc_pre(lanes8, Wm8, b_msg, ego_embed, W_fc, b_fc)

    w23 = W_msg[2:].reshape(-1)
    partials = _sc_agg(lane2, eil, eia, eax, eay, w23, N)

    W_dec_xy = W_dec.reshape(EMBED, FUT, 4)[:, :, :2].reshape(EMBED, FUT * 2)
    b_dec_xy = b_dec.reshape(FUT, 4)[:, :2].reshape(FUT * 2)
    y60 = y_hat.reshape(N, FUT * 2)
    yout60, pi = _tc_post(partials, h0, W_upd,
                          W_dec_xy, b_dec_xy, y60, W_pi.reshape(1, EMBED),
                          b_pi)
    return yout60.reshape(N, FUT, 2), pi
